# Initial kernel scaffold; baseline (speedup 1.0000x reference)
import jax
import jax.numpy as jnp
from jax.experimental import pallas as pl


def kernel(edge_attr, node_attr, edge_index, batch, eW1, eb1, eW2, eb2, eg, ebeta, nW1, nb1, nW2, nb2, ng, nbeta):
    raise NotImplementedError("write your pallas kernel here")


# speedup vs baseline: 1.6576x; 1.6576x over previous
import functools

import jax
import jax.numpy as jnp
from jax import lax
from jax.experimental import pallas as pl
from jax.experimental.pallas import tpu as pltpu
from jax.experimental.pallas import tpu_sc as plsc

N = 10000
E = 320000
D_N = 128
D_E = 16
LAT = 128

NC = 2
NS = 16
NW = NC * NS
EW = E // NW
CH = 80
NLOC = EW // CH
NPASS = 2
NRANGE = 7200
NRROW = NRANGE // 8

_SC_MESH = dict(core_axis_name="c", subcore_axis_name="s")


def _proj_body(na_ref, wr_ref, ws_ref, pr_ref, ps_ref):
    na = na_ref[...]
    pr_ref[...] = jnp.dot(na, wr_ref[...], preferred_element_type=jnp.float32)
    ps_ref[...] = jnp.dot(na, ws_ref[...], preferred_element_type=jnp.float32)


def _project(node_attr, wr, ws):
    bm = 1000
    grid = (N // bm,)
    return pl.pallas_call(
        _proj_body,
        grid=grid,
        in_specs=[
            pl.BlockSpec((bm, D_N), lambda i: (i, 0)),
            pl.BlockSpec((D_N, LAT), lambda i: (0, 0)),
            pl.BlockSpec((D_N, LAT), lambda i: (0, 0)),
        ],
        out_specs=[
            pl.BlockSpec((bm, LAT), lambda i: (i, 0)),
            pl.BlockSpec((bm, LAT), lambda i: (i, 0)),
        ],
        out_shape=[
            jax.ShapeDtypeStruct((N, LAT), jnp.float32),
            jax.ShapeDtypeStruct((N, LAT), jnp.float32),
        ],
    )(node_attr, wr, ws)


def _gather_body(pr_hbm, ps_hbm, row_hbm, col_hbm, out_hbm,
                 cidx, ridx, bufr, bufs, sem1, sem2):
    wid = lax.axis_index("s") * NC + lax.axis_index("c")

    def chunk(t, carry):
        base = wid * EW + t * CH
        pltpu.sync_copy(col_hbm.at[pl.ds(base, CH)], cidx)
        pltpu.sync_copy(row_hbm.at[pl.ds(base, CH)], ridx)
        cp1 = pltpu.async_copy(pr_hbm.at[cidx], bufr, sem1)
        cp2 = pltpu.async_copy(ps_hbm.at[ridx], bufs, sem2)
        cp1.wait()
        cp2.wait()

        def add_row(j, c2):
            for k in range(LAT // 16):
                sl = pl.ds(k * 16, 16)
                bufr[j, sl] = bufr[j, sl] + bufs[j, sl]
            return c2

        lax.fori_loop(0, CH, add_row, 0, unroll=2)
        pltpu.sync_copy(bufr, out_hbm.at[pl.ds(base, CH)])
        return carry

    lax.fori_loop(0, NLOC, chunk, 0)


def _gather_add(pr, ps, row, col):
    f = functools.partial(
        pl.kernel,
        mesh=plsc.VectorSubcoreMesh(**_SC_MESH),
        out_type=jax.ShapeDtypeStruct((E, LAT), jnp.float32),
        scratch_types=[
            pltpu.VMEM((CH,), jnp.int32),
            pltpu.VMEM((CH,), jnp.int32),
            pltpu.VMEM((CH, LAT), jnp.float32),
            pltpu.VMEM((CH, LAT), jnp.float32),
            pltpu.SemaphoreType.DMA,
            pltpu.SemaphoreType.DMA,
        ],
    )(_gather_body)
    return f(pr, ps, row, col)


def _edge_body(g_ref, ea_ref, w1_ref, b1_ref, w2_ref, b2_ref, gg_ref, gb_ref,
               out_ref):
    h = g_ref[...] + jnp.dot(ea_ref[...], w1_ref[...],
                             preferred_element_type=jnp.float32) + b1_ref[...]
    h = jnp.maximum(h, 0.0)
    h = jnp.dot(h, w2_ref[...], preferred_element_type=jnp.float32) + b2_ref[...]
    h = jnp.maximum(h, 0.0)
    mu = jnp.sum(h, axis=1, keepdims=True) * (1.0 / D_E)
    d = h - mu
    var = jnp.sum(d * d, axis=1, keepdims=True) * (1.0 / D_E)
    inv = lax.rsqrt(var + 1e-5)
    out_ref[...] = d * inv * gg_ref[...] + gb_ref[...]


def _edge_mlp(g, edge_attr, w1, b1, w2, b2, gg, gb):
    bm = 2000
    grid = (E // bm,)
    return pl.pallas_call(
        _edge_body,
        grid=grid,
        in_specs=[
            pl.BlockSpec((bm, LAT), lambda i: (i, 0)),
            pl.BlockSpec((bm, D_E), lambda i: (i, 0)),
            pl.BlockSpec((D_E, LAT), lambda i: (0, 0)),
            pl.BlockSpec((1, LAT), lambda i: (0, 0)),
            pl.BlockSpec((LAT, D_E), lambda i: (0, 0)),
            pl.BlockSpec((1, D_E), lambda i: (0, 0)),
            pl.BlockSpec((1, D_E), lambda i: (0, 0)),
            pl.BlockSpec((1, D_E), lambda i: (0, 0)),
        ],
        out_specs=pl.BlockSpec((bm, D_E), lambda i: (i, 0)),
        out_shape=jax.ShapeDtypeStruct((E, D_E), jnp.float32),
    )(g, edge_attr, w1, b1, w2, b2, gg, gb)


def _scatter_body(enew_hbm, col_hbm, zeros_hbm, out_hbm, cidx, rowbuf, acc):
    wid = lax.axis_index("s") * NC + lax.axis_index("c")
    iota = lax.iota(jnp.int32, 16)

    for p in range(NPASS):
        lo = p * NRANGE
        pltpu.sync_copy(zeros_hbm, acc)

        def chunk(t, carry):
            base = wid * EW + t * CH
            pltpu.sync_copy(col_hbm.at[pl.ds(base, CH)], cidx)
            pltpu.sync_copy(enew_hbm.at[pl.ds(base, CH)], rowbuf)
            for g in range(CH // 16):
                rloc16 = cidx[pl.ds(g * 16, 16)] - lo
                for j in range(16):
                    rl = jnp.broadcast_to(rloc16[j], (16,))
                    inr = plsc.bitcast(rl, jnp.uint32) < jnp.uint32(NRANGE)
                    rowi = lax.shift_right_arithmetic(rl, 3)
                    lane = lax.shift_left(lax.bitwise_and(rl, 7), 4) + iota
                    plsc.addupdate_scatter(acc, [rowi, lane],
                                           rowbuf[g * 16 + j, :], mask=inr)
            return carry

        lax.fori_loop(0, NLOC, chunk, 0)
        pltpu.sync_copy(acc, out_hbm.at[p * NW + wid])


def _scatter_sum(enew, col, zeros):
    f = functools.partial(
        pl.kernel,
        mesh=plsc.VectorSubcoreMesh(**_SC_MESH),
        compiler_params=pltpu.CompilerParams(needs_layout_passes=False),
        out_type=jax.ShapeDtypeStruct((NPASS * NW, NRROW, 128), jnp.float32),
        scratch_types=[
            pltpu.VMEM((CH,), jnp.int32),
            pltpu.VMEM((CH, D_E), jnp.float32),
            pltpu.VMEM((NRROW, 128), jnp.float32),
        ],
    )(_scatter_body)
    return f(enew, col, zeros)


def _node_body(na_ref, parts_ref, w1a_ref, w1b_ref, b1_ref,
               w2_ref, b2_ref, gg_ref, gb_ref, out_ref):
    r = jnp.sum(parts_ref[...], axis=0)
    h = (jnp.dot(na_ref[...], w1a_ref[...], preferred_element_type=jnp.float32)
         + jnp.dot(r, w1b_ref[...], preferred_element_type=jnp.float32)
         + b1_ref[...])
    h = jnp.maximum(h, 0.0)
    h = jnp.dot(h, w2_ref[...], preferred_element_type=jnp.float32) + b2_ref[...]
    h = jnp.maximum(h, 0.0)
    mu = jnp.sum(h, axis=1, keepdims=True) * (1.0 / D_N)
    d = h - mu
    var = jnp.sum(d * d, axis=1, keepdims=True) * (1.0 / D_N)
    inv = lax.rsqrt(var + 1e-5)
    out_ref[...] = d * inv * gg_ref[...] + gb_ref[...]


def _node_mlp(node_attr, parts, w1a, w1b, b1, w2, b2, gg, gb):
    bm = 400
    grid = (N // bm,)
    return pl.pallas_call(
        _node_body,
        grid=grid,
        in_specs=[
            pl.BlockSpec((bm, D_N), lambda i: (i, 0)),
            pl.BlockSpec((NW, bm, D_E), lambda i: (i // 18, i % 18, 0)),
            pl.BlockSpec((D_N, LAT), lambda i: (0, 0)),
            pl.BlockSpec((D_E, LAT), lambda i: (0, 0)),
            pl.BlockSpec((1, LAT), lambda i: (0, 0)),
            pl.BlockSpec((LAT, D_N), lambda i: (0, 0)),
            pl.BlockSpec((1, D_N), lambda i: (0, 0)),
            pl.BlockSpec((1, D_N), lambda i: (0, 0)),
            pl.BlockSpec((1, D_N), lambda i: (0, 0)),
        ],
        out_specs=pl.BlockSpec((bm, D_N), lambda i: (i, 0)),
        out_shape=jax.ShapeDtypeStruct((N, D_N), jnp.float32),
    )(node_attr, parts, w1a, w1b, b1, w2, b2, gg, gb)


def kernel(edge_attr, node_attr, edge_index, batch,
           eW1, eb1, eW2, eb2, eg, ebeta,
           nW1, nb1, nW2, nb2, ng, nbeta):
    row = edge_index[0]
    col = edge_index[1]

    pr, ps = _project(node_attr, eW1[:D_N], eW1[D_N:2 * D_N])
    g = _gather_add(pr, ps, row, col)
    enew = _edge_mlp(g, edge_attr, eW1[2 * D_N:],
                     eb1.reshape(1, LAT), eW2, eb2.reshape(1, D_E),
                     eg.reshape(1, D_E), ebeta.reshape(1, D_E))
    zeros = jnp.zeros((NRROW, 128), jnp.float32)
    parts = _scatter_sum(enew, col, zeros)
    parts = parts.reshape(NPASS * NW, NRANGE, D_E)
    nnew = _node_mlp(node_attr, parts,
                     nW1[:D_N], nW1[D_N:], nb1.reshape(1, LAT),
                     nW2, nb2.reshape(1, D_N),
                     ng.reshape(1, D_N), nbeta.reshape(1, D_N))
    return (enew, nnew)


# speedup vs baseline: 2.4866x; 1.5001x over previous
import functools

import jax
import jax.numpy as jnp
from jax import lax
from jax.experimental import pallas as pl
from jax.experimental.pallas import tpu as pltpu
from jax.experimental.pallas import tpu_sc as plsc

N = 10000
E = 320000
D_N = 128
D_E = 16
LAT = 128

NC = 2
NS = 16
NW = NC * NS
EW = E // NW
CH = 80
NLOC = EW // CH
NPASS = 2
NRANGE = 6400
NRROW = NRANGE // 8
SCH = 640
SNCH = E // SCH

_SC_MESH = dict(core_axis_name="c", subcore_axis_name="s")


def _proj_body(na_ref, wr_ref, ws_ref, pr_ref, ps_ref):
    na = na_ref[...]
    pr_ref[...] = jnp.dot(na, wr_ref[...], preferred_element_type=jnp.float32)
    ps_ref[...] = jnp.dot(na, ws_ref[...], preferred_element_type=jnp.float32)


def _project(node_attr, wr, ws):
    bm = 1000
    grid = (N // bm,)
    return pl.pallas_call(
        _proj_body,
        grid=grid,
        in_specs=[
            pl.BlockSpec((bm, D_N), lambda i: (i, 0)),
            pl.BlockSpec((D_N, LAT), lambda i: (0, 0)),
            pl.BlockSpec((D_N, LAT), lambda i: (0, 0)),
        ],
        out_specs=[
            pl.BlockSpec((bm, LAT), lambda i: (i, 0)),
            pl.BlockSpec((bm, LAT), lambda i: (i, 0)),
        ],
        out_shape=[
            jax.ShapeDtypeStruct((N, LAT), jnp.float32),
            jax.ShapeDtypeStruct((N, LAT), jnp.float32),
        ],
    )(node_attr, wr, ws)


def _gather_body(pr_hbm, ps_hbm, row_hbm, col_hbm, out_hbm,
                 cidx0, cidx1, ridx0, ridx1, bufr0, bufr1, bufs0, bufs1,
                 semi, semr, sems, semo):
    wid = lax.axis_index("s") * NC + lax.axis_index("c")
    cidx = (cidx0, cidx1)
    ridx = (ridx0, ridx1)
    bufr = (bufr0, bufr1)
    bufs = (bufs0, bufs1)

    def base_of(t):
        return wid * EW + t * CH

    def idx_copies(t, par):
        b = base_of(t)
        return (pltpu.make_async_copy(col_hbm.at[pl.ds(b, CH)], cidx[par],
                                      semi.at[par]),
                pltpu.make_async_copy(row_hbm.at[pl.ds(b, CH)], ridx[par],
                                      semi.at[par]))

    def gather_copies(par):
        return (pltpu.make_async_copy(pr_hbm.at[cidx[par]], bufr[par],
                                      semr.at[par]),
                pltpu.make_async_copy(ps_hbm.at[ridx[par]], bufs[par],
                                      sems.at[par]))

    def out_copy(t, par):
        return pltpu.make_async_copy(bufr[par],
                                     out_hbm.at[pl.ds(base_of(t), CH)],
                                     semo.at[par])

    pltpu.sync_copy(col_hbm.at[pl.ds(base_of(0), CH)], cidx[0])
    pltpu.sync_copy(row_hbm.at[pl.ds(base_of(0), CH)], ridx[0])
    for cp in gather_copies(0):
        cp.start()
    for cp in idx_copies(1, 1):
        cp.start()

    def step(k, carry):
        for par in (0, 1):
            nxt = 1 - par

            @pl.when(k % 2 == par)
            def _():
                for cp in gather_copies(par):
                    cp.wait()

                @pl.when(k + 2 < NLOC)
                def _():
                    for cp in idx_copies(k + 2, par):
                        cp.start()

                @pl.when(k + 1 < NLOC)
                def _():
                    @pl.when(k >= 1)
                    def _():
                        out_copy(k - 1, nxt).wait()
                    for cp in idx_copies(k + 1, nxt):
                        cp.wait()
                    for cp in gather_copies(nxt):
                        cp.start()

                def add_row(j, c2):
                    for q in range(LAT // 16):
                        sl = pl.ds(q * 16, 16)
                        bufr[par][j, sl] = bufr[par][j, sl] + bufs[par][j, sl]
                    return c2

                lax.fori_loop(0, CH, add_row, 0, unroll=2)
                out_copy(k, par).start()

        return carry

    lax.fori_loop(0, NLOC, step, 0)
    out_copy(NLOC - 2, (NLOC - 2) % 2).wait()
    out_copy(NLOC - 1, (NLOC - 1) % 2).wait()


def _gather_add(pr, ps, row, col):
    f = functools.partial(
        pl.kernel,
        mesh=plsc.VectorSubcoreMesh(**_SC_MESH),
        out_type=jax.ShapeDtypeStruct((E, LAT), jnp.float32),
        scratch_types=[
            pltpu.VMEM((CH,), jnp.int32),
            pltpu.VMEM((CH,), jnp.int32),
            pltpu.VMEM((CH,), jnp.int32),
            pltpu.VMEM((CH,), jnp.int32),
            pltpu.VMEM((CH, LAT), jnp.float32),
            pltpu.VMEM((CH, LAT), jnp.float32),
            pltpu.VMEM((CH, LAT), jnp.float32),
            pltpu.VMEM((CH, LAT), jnp.float32),
            pltpu.SemaphoreType.DMA((2,)),
            pltpu.SemaphoreType.DMA((2,)),
            pltpu.SemaphoreType.DMA((2,)),
            pltpu.SemaphoreType.DMA((2,)),
        ],
    )(_gather_body)
    return f(pr, ps, row, col)


def _edge_body(g_ref, ea_ref, w1_ref, b1_ref, w2_ref, b2_ref, gg_ref, gb_ref,
               out_ref):
    h = g_ref[...] + jnp.dot(ea_ref[...], w1_ref[...],
                             preferred_element_type=jnp.float32) + b1_ref[...]
    h = jnp.maximum(h, 0.0)
    h = jnp.dot(h, w2_ref[...], preferred_element_type=jnp.float32) + b2_ref[...]
    h = jnp.maximum(h, 0.0)
    mu = jnp.sum(h, axis=1, keepdims=True) * (1.0 / D_E)
    d = h - mu
    var = jnp.sum(d * d, axis=1, keepdims=True) * (1.0 / D_E)
    inv = lax.rsqrt(var + 1e-5)
    out_ref[...] = d * inv * gg_ref[...] + gb_ref[...]


def _edge_mlp(g, edge_attr, w1, b1, w2, b2, gg, gb):
    bm = 2000
    grid = (E // bm,)
    return pl.pallas_call(
        _edge_body,
        grid=grid,
        in_specs=[
            pl.BlockSpec((bm, LAT), lambda i: (i, 0)),
            pl.BlockSpec((bm, D_E), lambda i: (i, 0)),
            pl.BlockSpec((D_E, LAT), lambda i: (0, 0)),
            pl.BlockSpec((1, LAT), lambda i: (0, 0)),
            pl.BlockSpec((LAT, D_E), lambda i: (0, 0)),
            pl.BlockSpec((1, D_E), lambda i: (0, 0)),
            pl.BlockSpec((1, D_E), lambda i: (0, 0)),
            pl.BlockSpec((1, D_E), lambda i: (0, 0)),
        ],
        out_specs=pl.BlockSpec((bm, D_E), lambda i: (i, 0)),
        out_shape=jax.ShapeDtypeStruct((E, D_E), jnp.float32),
    )(g, edge_attr, w1, b1, w2, b2, gg, gb)


def _scatter_body(enew_hbm, col_hbm, zeros_hbm, out_hbm,
                  cidx0, cidx1, rbuf0, rbuf1, acc, semi, semd):
    wid = lax.axis_index("s") * NC + lax.axis_index("c")
    iota = lax.iota(jnp.int32, 16)
    lane_consts = [jnp.int32(m * 16) + iota for m in range(8)]
    nloc = jnp.where(wid < SNCH - NW * (SNCH // NW), SNCH // NW + 1,
                     SNCH // NW)
    cidx = (cidx0, cidx1)
    rbuf = (rbuf0, rbuf1)

    def load_copies(t, par):
        c = t * NW + wid
        return (pltpu.make_async_copy(col_hbm.at[pl.ds(c * SCH, SCH)],
                                      cidx[par], semi.at[par]),
                pltpu.make_async_copy(enew_hbm.at[pl.ds(c * (SCH // 8),
                                                        SCH // 8)],
                                      rbuf[par], semd.at[par]))

    for p in range(NPASS):
        lo = p * NRANGE
        pltpu.sync_copy(zeros_hbm, acc)
        for cp in load_copies(0, 0):
            cp.start()

        def step(t, carry):
            for par in (0, 1):
                nxt = 1 - par

                @pl.when(t % 2 == par)
                def _():
                    for cp in load_copies(t, par):
                        cp.wait()

                    @pl.when(t + 1 < nloc)
                    def _():
                        for cp in load_copies(t + 1, nxt):
                            cp.start()

                    def group(g, c2):
                        rloc16 = cidx[par][pl.ds(g * 16, 16)] - lo
                        srca = jnp.broadcast_to(2 * g, (16,)).astype(jnp.int32)
                        srcrow = [srca, srca + 1]
                        for j in range(16):
                            rl = jnp.broadcast_to(rloc16[j], (16,))
                            inr = (plsc.bitcast(rl, jnp.uint32)
                                   < jnp.uint32(NRANGE))
                            rowi = lax.shift_right_arithmetic(rl, 3)
                            lane = (lax.shift_left(lax.bitwise_and(rl, 7), 4)
                                    + iota)
                            vals = plsc.load_gather(
                                rbuf[par], [srcrow[j // 8], lane_consts[j % 8]])
                            plsc.addupdate_scatter(acc, [rowi, lane], vals,
                                                   mask=inr)
                        return c2

                    lax.fori_loop(0, SCH // 16, group, 0)

            return carry

        lax.fori_loop(0, nloc, step, 0)
        pltpu.sync_copy(acc, out_hbm.at[p * NW + wid])


def _scatter_sum(enew128, col, zeros):
    f = functools.partial(
        pl.kernel,
        mesh=plsc.VectorSubcoreMesh(**_SC_MESH),
        compiler_params=pltpu.CompilerParams(needs_layout_passes=False),
        out_type=jax.ShapeDtypeStruct((NPASS * NW, NRROW, 128), jnp.float32),
        scratch_types=[
            pltpu.VMEM((SCH,), jnp.int32),
            pltpu.VMEM((SCH,), jnp.int32),
            pltpu.VMEM((SCH // 8, 128), jnp.float32),
            pltpu.VMEM((SCH // 8, 128), jnp.float32),
            pltpu.VMEM((NRROW, 128), jnp.float32),
            pltpu.SemaphoreType.DMA((2,)),
            pltpu.SemaphoreType.DMA((2,)),
        ],
    )(_scatter_body)
    return f(enew128, col, zeros)


def _node_body(na_ref, parts_ref, w1a_ref, w1b_ref, b1_ref,
               w2_ref, b2_ref, gg_ref, gb_ref, out_ref):
    r = jnp.sum(parts_ref[...], axis=0)
    h = (jnp.dot(na_ref[...], w1a_ref[...], preferred_element_type=jnp.float32)
         + jnp.dot(r, w1b_ref[...], preferred_element_type=jnp.float32)
         + b1_ref[...])
    h = jnp.maximum(h, 0.0)
    h = jnp.dot(h, w2_ref[...], preferred_element_type=jnp.float32) + b2_ref[...]
    h = jnp.maximum(h, 0.0)
    mu = jnp.sum(h, axis=1, keepdims=True) * (1.0 / D_N)
    d = h - mu
    var = jnp.sum(d * d, axis=1, keepdims=True) * (1.0 / D_N)
    inv = lax.rsqrt(var + 1e-5)
    out_ref[...] = d * inv * gg_ref[...] + gb_ref[...]


def _node_mlp(node_attr, parts, w1a, w1b, b1, w2, b2, gg, gb):
    bm = 400
    grid = (N // bm,)
    return pl.pallas_call(
        _node_body,
        grid=grid,
        in_specs=[
            pl.BlockSpec((bm, D_N), lambda i: (i, 0)),
            pl.BlockSpec((NW, bm, D_E), lambda i: (i // 16, i % 16, 0)),
            pl.BlockSpec((D_N, LAT), lambda i: (0, 0)),
            pl.BlockSpec((D_E, LAT), lambda i: (0, 0)),
            pl.BlockSpec((1, LAT), lambda i: (0, 0)),
            pl.BlockSpec((LAT, D_N), lambda i: (0, 0)),
            pl.BlockSpec((1, D_N), lambda i: (0, 0)),
            pl.BlockSpec((1, D_N), lambda i: (0, 0)),
            pl.BlockSpec((1, D_N), lambda i: (0, 0)),
        ],
        out_specs=pl.BlockSpec((bm, D_N), lambda i: (i, 0)),
        out_shape=jax.ShapeDtypeStruct((N, D_N), jnp.float32),
    )(node_attr, parts, w1a, w1b, b1, w2, b2, gg, gb)


def kernel(edge_attr, node_attr, edge_index, batch,
           eW1, eb1, eW2, eb2, eg, ebeta,
           nW1, nb1, nW2, nb2, ng, nbeta):
    row = edge_index[0]
    col = edge_index[1]

    pr, ps = _project(node_attr, eW1[:D_N], eW1[D_N:2 * D_N])
    g = _gather_add(pr, ps, row, col)
    enew = _edge_mlp(g, edge_attr, eW1[2 * D_N:],
                     eb1.reshape(1, LAT), eW2, eb2.reshape(1, D_E),
                     eg.reshape(1, D_E), ebeta.reshape(1, D_E))
    zeros = jnp.zeros((NRROW, 128), jnp.float32)
    parts = _scatter_sum(enew.reshape(E // 8, 128), col, zeros)
    parts = parts.reshape(NPASS * NW, NRANGE, D_E)
    nnew = _node_mlp(node_attr, parts,
                     nW1[:D_N], nW1[D_N:], nb1.reshape(1, LAT),
                     nW2, nb2.reshape(1, D_N),
                     ng.reshape(1, D_N), nbeta.reshape(1, D_N))
    return (enew, nnew)


# speedup vs baseline: 2.6071x; 1.0485x over previous
import functools

import jax
import jax.numpy as jnp
from jax import lax
from jax.experimental import pallas as pl
from jax.experimental.pallas import tpu as pltpu
from jax.experimental.pallas import tpu_sc as plsc

N = 10000
E = 320000
D_N = 128
D_E = 16
LAT = 128

NC = 2
NS = 16
NW = NC * NS
EW = E // NW
CH = 80
NLOC = EW // CH
NPASS = 2
NRANGE = 6400
NRROW = NRANGE // 8
SCH = 640
SNCH = E // SCH

_SC_MESH = dict(core_axis_name="c", subcore_axis_name="s")


def _proj_body(na_ref, wr_ref, ws_ref, pr_ref, ps_ref):
    na = na_ref[...]
    pr_ref[...] = jnp.dot(na, wr_ref[...], preferred_element_type=jnp.float32)
    ps_ref[...] = jnp.dot(na, ws_ref[...], preferred_element_type=jnp.float32)


def _project(node_attr, wr, ws):
    bm = 1000
    grid = (N // bm,)
    return pl.pallas_call(
        _proj_body,
        grid=grid,
        in_specs=[
            pl.BlockSpec((bm, D_N), lambda i: (i, 0)),
            pl.BlockSpec((D_N, LAT), lambda i: (0, 0)),
            pl.BlockSpec((D_N, LAT), lambda i: (0, 0)),
        ],
        out_specs=[
            pl.BlockSpec((bm, LAT), lambda i: (i, 0)),
            pl.BlockSpec((bm, LAT), lambda i: (i, 0)),
        ],
        out_shape=[
            jax.ShapeDtypeStruct((N, LAT), jnp.float32),
            jax.ShapeDtypeStruct((N, LAT), jnp.float32),
        ],
    )(node_attr, wr, ws)


def _gather_body(pr_hbm, ps_hbm, row_hbm, col_hbm, out_hbm,
                 cidx0, cidx1, ridx0, ridx1, bufr0, bufr1, bufs0, bufs1,
                 semi, semr, sems, semo):
    wid = lax.axis_index("s") * NC + lax.axis_index("c")
    cidx = (cidx0, cidx1)
    ridx = (ridx0, ridx1)
    bufr = (bufr0, bufr1)
    bufs = (bufs0, bufs1)

    def base_of(t):
        return wid * EW + t * CH

    def idx_copies(t, par):
        b = base_of(t)
        return (pltpu.make_async_copy(col_hbm.at[pl.ds(b, CH)], cidx[par],
                                      semi.at[par]),
                pltpu.make_async_copy(row_hbm.at[pl.ds(b, CH)], ridx[par],
                                      semi.at[par]))

    def gather_copies(par):
        return (pltpu.make_async_copy(pr_hbm.at[cidx[par]], bufr[par],
                                      semr.at[par]),
                pltpu.make_async_copy(ps_hbm.at[ridx[par]], bufs[par],
                                      sems.at[par]))

    def out_copy(t, par):
        return pltpu.make_async_copy(bufr[par],
                                     out_hbm.at[pl.ds(base_of(t), CH)],
                                     semo.at[par])

    pltpu.sync_copy(col_hbm.at[pl.ds(base_of(0), CH)], cidx[0])
    pltpu.sync_copy(row_hbm.at[pl.ds(base_of(0), CH)], ridx[0])
    for cp in gather_copies(0):
        cp.start()
    for cp in idx_copies(1, 1):
        cp.start()

    def step(k, carry):
        for par in (0, 1):
            nxt = 1 - par

            @pl.when(k % 2 == par)
            def _():
                for cp in gather_copies(par):
                    cp.wait()

                @pl.when(k + 2 < NLOC)
                def _():
                    for cp in idx_copies(k + 2, par):
                        cp.start()

                @pl.when(k + 1 < NLOC)
                def _():
                    @pl.when(k >= 1)
                    def _():
                        out_copy(k - 1, nxt).wait()
                    for cp in idx_copies(k + 1, nxt):
                        cp.wait()
                    for cp in gather_copies(nxt):
                        cp.start()

                def add_row(j, c2):
                    for q in range(LAT // 16):
                        sl = pl.ds(q * 16, 16)
                        bufr[par][j, sl] = bufr[par][j, sl] + bufs[par][j, sl]
                    return c2

                lax.fori_loop(0, CH, add_row, 0, unroll=2)
                out_copy(k, par).start()

        return carry

    lax.fori_loop(0, NLOC, step, 0)
    out_copy(NLOC - 2, (NLOC - 2) % 2).wait()
    out_copy(NLOC - 1, (NLOC - 1) % 2).wait()


def _gather_add(pr, ps, row, col):
    f = functools.partial(
        pl.kernel,
        mesh=plsc.VectorSubcoreMesh(**_SC_MESH),
        out_type=jax.ShapeDtypeStruct((E, LAT), jnp.float32),
        scratch_types=[
            pltpu.VMEM((CH,), jnp.int32),
            pltpu.VMEM((CH,), jnp.int32),
            pltpu.VMEM((CH,), jnp.int32),
            pltpu.VMEM((CH,), jnp.int32),
            pltpu.VMEM((CH, LAT), jnp.float32),
            pltpu.VMEM((CH, LAT), jnp.float32),
            pltpu.VMEM((CH, LAT), jnp.float32),
            pltpu.VMEM((CH, LAT), jnp.float32),
            pltpu.SemaphoreType.DMA((2,)),
            pltpu.SemaphoreType.DMA((2,)),
            pltpu.SemaphoreType.DMA((2,)),
            pltpu.SemaphoreType.DMA((2,)),
        ],
    )(_gather_body)
    return f(pr, ps, row, col)


def _edge_body(g_ref, ea_ref, w1_ref, b1_ref, w2_ref, b2_ref, gg_ref, gb_ref,
               out_ref):
    h = g_ref[...] + jnp.dot(ea_ref[...], w1_ref[...],
                             preferred_element_type=jnp.float32) + b1_ref[...]
    h = jnp.maximum(h, 0.0)
    h = jnp.dot(h, w2_ref[...], preferred_element_type=jnp.float32) + b2_ref[...]
    h = jnp.maximum(h, 0.0)
    avg = jnp.full((D_E, D_E), 1.0 / D_E, jnp.float32)
    mu = jnp.dot(h, avg, preferred_element_type=jnp.float32)
    d = h - mu
    var = jnp.dot(d * d, avg, preferred_element_type=jnp.float32)
    inv = lax.rsqrt(var + 1e-5)
    out_ref[...] = d * inv * gg_ref[...] + gb_ref[...]


def _edge_mlp(g, edge_attr, w1, b1, w2, b2, gg, gb):
    bm = 4000
    grid = (E // bm,)
    return pl.pallas_call(
        _edge_body,
        grid=grid,
        in_specs=[
            pl.BlockSpec((bm, LAT), lambda i: (i, 0)),
            pl.BlockSpec((bm, D_E), lambda i: (i, 0)),
            pl.BlockSpec((D_E, LAT), lambda i: (0, 0)),
            pl.BlockSpec((1, LAT), lambda i: (0, 0)),
            pl.BlockSpec((LAT, D_E), lambda i: (0, 0)),
            pl.BlockSpec((1, D_E), lambda i: (0, 0)),
            pl.BlockSpec((1, D_E), lambda i: (0, 0)),
            pl.BlockSpec((1, D_E), lambda i: (0, 0)),
        ],
        out_specs=pl.BlockSpec((bm, D_E), lambda i: (i, 0)),
        out_shape=jax.ShapeDtypeStruct((E, D_E), jnp.float32),
    )(g, edge_attr, w1, b1, w2, b2, gg, gb)


def _scatter_body(enew_hbm, col_hbm, zeros_hbm, out_hbm,
                  cidx0, cidx1, rbuf0, rbuf1, acc, semi, semd):
    wid = lax.axis_index("s") * NC + lax.axis_index("c")
    iota = lax.iota(jnp.int32, 16)
    lane_consts = [jnp.int32(m * 16) + iota for m in range(8)]
    nloc = jnp.where(wid < SNCH - NW * (SNCH // NW), SNCH // NW + 1,
                     SNCH // NW)
    cidx = (cidx0, cidx1)
    rbuf = (rbuf0, rbuf1)

    def load_copies(t, par):
        c = t * NW + wid
        return (pltpu.make_async_copy(col_hbm.at[pl.ds(c * SCH, SCH)],
                                      cidx[par], semi.at[par]),
                pltpu.make_async_copy(enew_hbm.at[pl.ds(c * (SCH // 8),
                                                        SCH // 8)],
                                      rbuf[par], semd.at[par]))

    for p in range(NPASS):
        lo = p * NRANGE
        pltpu.sync_copy(zeros_hbm, acc)
        for cp in load_copies(0, 0):
            cp.start()

        def step(t, carry):
            for par in (0, 1):
                nxt = 1 - par

                @pl.when(t % 2 == par)
                def _():
                    for cp in load_copies(t, par):
                        cp.wait()

                    @pl.when(t + 1 < nloc)
                    def _():
                        for cp in load_copies(t + 1, nxt):
                            cp.start()

                    def group(g, c2):
                        rloc16 = cidx[par][pl.ds(g * 16, 16)] - lo
                        srca = jnp.broadcast_to(2 * g, (16,)).astype(jnp.int32)
                        srcrow = [srca, srca + 1]
                        for j in range(16):
                            rl = jnp.broadcast_to(rloc16[j], (16,))
                            inr = (plsc.bitcast(rl, jnp.uint32)
                                   < jnp.uint32(NRANGE))
                            rowi = lax.shift_right_arithmetic(rl, 3)
                            lane = (lax.shift_left(lax.bitwise_and(rl, 7), 4)
                                    + iota)
                            vals = plsc.load_gather(
                                rbuf[par], [srcrow[j // 8], lane_consts[j % 8]])
                            plsc.addupdate_scatter(acc, [rowi, lane], vals,
                                                   mask=inr)
                        return c2

                    lax.fori_loop(0, SCH // 16, group, 0)

            return carry

        lax.fori_loop(0, nloc, step, 0)
        pltpu.sync_copy(acc, out_hbm.at[p * NW + wid])


def _scatter_sum(enew128, col, zeros):
    f = functools.partial(
        pl.kernel,
        mesh=plsc.VectorSubcoreMesh(**_SC_MESH),
        compiler_params=pltpu.CompilerParams(needs_layout_passes=False),
        out_type=jax.ShapeDtypeStruct((NPASS * NW, NRROW, 128), jnp.float32),
        scratch_types=[
            pltpu.VMEM((SCH,), jnp.int32),
            pltpu.VMEM((SCH,), jnp.int32),
            pltpu.VMEM((SCH // 8, 128), jnp.float32),
            pltpu.VMEM((SCH // 8, 128), jnp.float32),
            pltpu.VMEM((NRROW, 128), jnp.float32),
            pltpu.SemaphoreType.DMA((2,)),
            pltpu.SemaphoreType.DMA((2,)),
        ],
    )(_scatter_body)
    return f(enew128, col, zeros)


def _node_body(na_ref, parts_ref, w1a_ref, w1b_ref, b1_ref,
               w2_ref, b2_ref, gg_ref, gb_ref, out_ref):
    r = jnp.sum(parts_ref[...], axis=0)
    h = (jnp.dot(na_ref[...], w1a_ref[...], preferred_element_type=jnp.float32)
         + jnp.dot(r, w1b_ref[...], preferred_element_type=jnp.float32)
         + b1_ref[...])
    h = jnp.maximum(h, 0.0)
    h = jnp.dot(h, w2_ref[...], preferred_element_type=jnp.float32) + b2_ref[...]
    h = jnp.maximum(h, 0.0)
    avg = jnp.full((D_N, D_N), 1.0 / D_N, jnp.float32)
    mu = jnp.dot(h, avg, preferred_element_type=jnp.float32)
    d = h - mu
    var = jnp.dot(d * d, avg, preferred_element_type=jnp.float32)
    inv = lax.rsqrt(var + 1e-5)
    out_ref[...] = d * inv * gg_ref[...] + gb_ref[...]


def _node_mlp(node_attr, parts, w1a, w1b, b1, w2, b2, gg, gb):
    bm = 400
    grid = (N // bm,)
    return pl.pallas_call(
        _node_body,
        grid=grid,
        in_specs=[
            pl.BlockSpec((bm, D_N), lambda i: (i, 0)),
            pl.BlockSpec((NW, bm, D_E), lambda i: (i // 16, i % 16, 0)),
            pl.BlockSpec((D_N, LAT), lambda i: (0, 0)),
            pl.BlockSpec((D_E, LAT), lambda i: (0, 0)),
            pl.BlockSpec((1, LAT), lambda i: (0, 0)),
            pl.BlockSpec((LAT, D_N), lambda i: (0, 0)),
            pl.BlockSpec((1, D_N), lambda i: (0, 0)),
            pl.BlockSpec((1, D_N), lambda i: (0, 0)),
            pl.BlockSpec((1, D_N), lambda i: (0, 0)),
        ],
        out_specs=pl.BlockSpec((bm, D_N), lambda i: (i, 0)),
        out_shape=jax.ShapeDtypeStruct((N, D_N), jnp.float32),
    )(node_attr, parts, w1a, w1b, b1, w2, b2, gg, gb)


def kernel(edge_attr, node_attr, edge_index, batch,
           eW1, eb1, eW2, eb2, eg, ebeta,
           nW1, nb1, nW2, nb2, ng, nbeta):
    row = edge_index[0]
    col = edge_index[1]

    pr, ps = _project(node_attr, eW1[:D_N], eW1[D_N:2 * D_N])
    g = _gather_add(pr, ps, row, col)
    enew = _edge_mlp(g, edge_attr, eW1[2 * D_N:],
                     eb1.reshape(1, LAT), eW2, eb2.reshape(1, D_E),
                     eg.reshape(1, D_E), ebeta.reshape(1, D_E))
    zeros = jnp.zeros((NRROW, 128), jnp.float32)
    parts = _scatter_sum(enew.reshape(E // 8, 128), col, zeros)
    parts = parts.reshape(NPASS * NW, NRANGE, D_E)
    nnew = _node_mlp(node_attr, parts,
                     nW1[:D_N], nW1[D_N:], nb1.reshape(1, LAT),
                     nW2, nb2.reshape(1, D_N),
                     ng.reshape(1, D_N), nbeta.reshape(1, D_N))
    return (enew, nnew)
